# baseline (speedup 1.0000x reference)
import functools

import jax
import jax.numpy as jnp
from jax import lax
from jax.experimental import pallas as pl
from jax.experimental.pallas import tpu as pltpu
from jax.experimental.pallas import tpu_sc as plsc

_B = 16384
_D = 64


def _sc_gather(idx, t_mlp, t_mf):
    info = plsc.get_sparse_core_info()
    nc, ns = info.num_cores, info.num_subcores
    nw = nc * ns
    bpw = _B // nw

    mesh = plsc.VectorSubcoreMesh(core_axis_name="c", subcore_axis_name="s")

    @functools.partial(
        pl.kernel,
        out_type=(
            jax.ShapeDtypeStruct((_B, _D), jnp.float32),
            jax.ShapeDtypeStruct((_B, _D), jnp.float32),
        ),
        mesh=mesh,
        scratch_types=[
            pltpu.VMEM((bpw,), jnp.int32),
            pltpu.SemaphoreType.DMA,
        ],
    )
    def k(idx_hbm, t1_hbm, t2_hbm, o1_hbm, o2_hbm, idx_v, sem):
        wid = lax.axis_index("s") * nc + lax.axis_index("c")
        base = wid * bpw
        pltpu.sync_copy(idx_hbm.at[pl.ds(base, bpw)], idx_v)

        def body(g, _):
            v = idx_v[pl.ds(g * 16, 16)] - 1
            for j in range(16):
                r = v[j]
                i = g * 16 + j
                pltpu.async_copy(
                    t1_hbm.at[pl.ds(r, 1), :],
                    o1_hbm.at[pl.ds(base + i, 1), :], sem)
                pltpu.async_copy(
                    t2_hbm.at[pl.ds(r, 1), :],
                    o2_hbm.at[pl.ds(base + i, 1), :], sem)
            return _

        lax.fori_loop(0, bpw // 16, body, 0)
        pltpu.make_async_copy(
            t1_hbm.at[pl.ds(0, 2 * bpw)], o1_hbm.at[pl.ds(0, 2 * bpw)],
            sem).wait()

    return k(idx, t_mlp, t_mf)


def _tc_body(g1_ref, g2_ref, um_ref, uf_ref, w1_ref, b1_ref, w2_ref,
             b2_ref, w3_ref, b3_ref, wa_ref, ba_ref, o_ref):
    w1 = w1_ref[...]
    h1 = jnp.dot(g1_ref[...], w1[_D:, :], preferred_element_type=jnp.float32)
    h1 = h1 + jnp.dot(um_ref[...], w1[:_D, :],
                      preferred_element_type=jnp.float32)
    h1 = jnp.maximum(h1 + b1_ref[...], 0.0)
    h2 = jnp.maximum(
        jnp.dot(h1, w2_ref[...], preferred_element_type=jnp.float32)
        + b2_ref[...], 0.0)
    h3 = jnp.maximum(
        jnp.dot(h2, w3_ref[...], preferred_element_type=jnp.float32)
        + b3_ref[...], 0.0)
    wa = wa_ref[...]
    s = jnp.dot(h3, wa[:16, :], preferred_element_type=jnp.float32)
    s = s + jnp.dot(g2_ref[...] * uf_ref[...], wa[16:, :],
                    preferred_element_type=jnp.float32)
    o_ref[...] = jax.nn.sigmoid(s + ba_ref[...])[:, 0]


def _tc_mlp(g1, g2, u_mlp, u_mf, w1t, b1, w2t, b2, w3t, b3, wat, ba):
    blk = 2048
    grid = _B // blk
    fixed = lambda shape: pl.BlockSpec(shape, lambda i: (0,) * len(shape))
    return pl.pallas_call(
        _tc_body,
        grid=(grid,),
        in_specs=[
            pl.BlockSpec((blk, _D), lambda i: (i, 0)),
            pl.BlockSpec((blk, _D), lambda i: (i, 0)),
            fixed((1, _D)),
            fixed((1, _D)),
            fixed((2 * _D, _D)),
            fixed((1, _D)),
            fixed((_D, 32)),
            fixed((1, 32)),
            fixed((32, 16)),
            fixed((1, 16)),
            fixed((16 + _D, 1)),
            fixed((1, 1)),
        ],
        out_specs=pl.BlockSpec((blk,), lambda i: (i,)),
        out_shape=jax.ShapeDtypeStruct((_B,), jnp.float32),
    )(g1, g2, u_mlp, u_mf, w1t, b1, w2t, b2, w3t, b3, wat, ba)


def kernel(item_indices, emb_user_mlp, emb_item_mlp, emb_user_mf,
           emb_item_mf, W1, b1, W2, b2, W3, b3, Wa, ba):
    g1, g2 = _sc_gather(item_indices, emb_item_mlp, emb_item_mf)
    return _tc_mlp(
        g1, g2, emb_user_mlp, emb_user_mf,
        W1.T, b1.reshape(1, -1), W2.T, b2.reshape(1, -1),
        W3.T, b3.reshape(1, -1), Wa.T, ba.reshape(1, 1))


# speedup vs baseline: 1.6696x; 1.6696x over previous
import functools

import jax
import jax.numpy as jnp
from jax import lax
from jax.experimental import pallas as pl
from jax.experimental.pallas import tpu as pltpu
from jax.experimental.pallas import tpu_sc as plsc

_B = 16384
_D = 64


def _sc_gather(idx, t_mlp, t_mf):
    info = plsc.get_sparse_core_info()
    nc, ns = info.num_cores, info.num_subcores
    nw = nc * ns
    bpw = _B // nw

    mesh = plsc.VectorSubcoreMesh(core_axis_name="c", subcore_axis_name="s")

    @functools.partial(
        pl.kernel,
        out_type=(
            jax.ShapeDtypeStruct((_B, _D), jnp.float32),
            jax.ShapeDtypeStruct((_B, _D), jnp.float32),
        ),
        mesh=mesh,
        scratch_types=[
            pltpu.VMEM((bpw,), jnp.int32),
            pltpu.VMEM((16, _D), jnp.float32),
            pltpu.VMEM((16, _D), jnp.float32),
            pltpu.VMEM((16, _D), jnp.float32),
            pltpu.VMEM((16, _D), jnp.float32),
            pltpu.SemaphoreType.DMA,
            pltpu.SemaphoreType.DMA,
            pltpu.SemaphoreType.DMA,
            pltpu.SemaphoreType.DMA,
        ],
    )
    def k(idx_hbm, t1_hbm, t2_hbm, o1_hbm, o2_hbm, ia_v,
          a1_v, a2_v, b1_v, b2_v, sa0, sa1, sb0, sb1):
        wid = lax.axis_index("s") * nc + lax.axis_index("c")
        base = wid * bpw
        pltpu.sync_copy(idx_hbm.at[pl.ds(base, bpw)], ia_v)
        for g in range(bpw // 16):
            sl = pl.ds(g * 16, 16)
            ia_v[sl] = ia_v[sl] - 1

        def fire(g, bufs):
            g1_v, g2_v, s0, s1 = bufs
            v = ia_v[pl.ds(g * 16, 16)]
            for j in range(16):
                pltpu.async_copy(
                    t1_hbm.at[pl.ds(v[j], 1), :],
                    g1_v.at[pl.ds(j, 1), :], s0)
                pltpu.async_copy(
                    t2_hbm.at[pl.ds(v[j], 1), :],
                    g2_v.at[pl.ds(j, 1), :], s1)

        def drain_out(g, bufs):
            g1_v, g2_v, s0, s1 = bufs
            pltpu.make_async_copy(
                t1_hbm.at[pl.ds(0, 16), :], g1_v, s0).wait()
            pltpu.sync_copy(g1_v, o1_hbm.at[pl.ds(base + g * 16, 16), :])
            pltpu.make_async_copy(
                t2_hbm.at[pl.ds(0, 16), :], g2_v, s1).wait()
            pltpu.sync_copy(g2_v, o2_hbm.at[pl.ds(base + g * 16, 16), :])

        ba = (a1_v, a2_v, sa0, sa1)
        bb = (b1_v, b2_v, sb0, sb1)
        fire(0, ba)

        def step(h, _):
            g = 2 * h
            fire(g + 1, bb)
            drain_out(g, ba)
            fire(g + 2, ba)
            drain_out(g + 1, bb)
            return _

        lax.fori_loop(0, bpw // 32 - 1, step, 0)
        g_last = bpw // 16 - 2
        fire(g_last + 1, bb)
        drain_out(g_last, ba)
        drain_out(g_last + 1, bb)

    return k(idx, t_mlp, t_mf)


def _tc_body(g1_ref, g2_ref, um_ref, uf_ref, w1_ref, b1_ref, w2_ref,
             b2_ref, w3_ref, b3_ref, wa_ref, ba_ref, o_ref):
    w1 = w1_ref[...]
    h1 = jnp.dot(g1_ref[...], w1[_D:, :],
                 preferred_element_type=jnp.float32)
    h1 = h1 + jnp.dot(um_ref[...], w1[:_D, :],
                      preferred_element_type=jnp.float32)
    h1 = jnp.maximum(h1 + b1_ref[...], 0.0)
    h2 = jnp.maximum(
        jnp.dot(h1, w2_ref[...], preferred_element_type=jnp.float32)
        + b2_ref[...], 0.0)
    h3 = jnp.maximum(
        jnp.dot(h2, w3_ref[...], preferred_element_type=jnp.float32)
        + b3_ref[...], 0.0)
    wa = wa_ref[...]
    s = jnp.dot(h3, wa[:16, :], preferred_element_type=jnp.float32)
    s = s + jnp.dot(g2_ref[...] * uf_ref[...], wa[16:, :],
                    preferred_element_type=jnp.float32)
    o_ref[...] = jax.nn.sigmoid(s + ba_ref[...])[:, 0]


def _tc_mlp(g1, g2, u_mlp, u_mf, w1t, b1, w2t, b2, w3t, b3, wat, ba):
    blk = 2048
    grid = _B // blk
    fixed = lambda shape: pl.BlockSpec(shape, lambda i: (0,) * len(shape))
    return pl.pallas_call(
        _tc_body,
        grid=(grid,),
        in_specs=[
            pl.BlockSpec((blk, _D), lambda i: (i, 0)),
            pl.BlockSpec((blk, _D), lambda i: (i, 0)),
            fixed((1, _D)),
            fixed((1, _D)),
            fixed((2 * _D, _D)),
            fixed((1, _D)),
            fixed((_D, 32)),
            fixed((1, 32)),
            fixed((32, 16)),
            fixed((1, 16)),
            fixed((16 + _D, 1)),
            fixed((1, 1)),
        ],
        out_specs=pl.BlockSpec((blk,), lambda i: (i,)),
        out_shape=jax.ShapeDtypeStruct((_B,), jnp.float32),
    )(g1, g2, u_mlp, u_mf, w1t, b1, w2t, b2, w3t, b3, wat, ba)


def kernel(item_indices, emb_user_mlp, emb_item_mlp, emb_user_mf,
           emb_item_mf, W1, b1, W2, b2, W3, b3, Wa, ba):
    g1, g2 = _sc_gather(item_indices, emb_item_mlp, emb_item_mf)
    return _tc_mlp(
        g1, g2, emb_user_mlp, emb_user_mf,
        W1.T, b1.reshape(1, -1), W2.T, b2.reshape(1, -1),
        W3.T, b3.reshape(1, -1), Wa.T, ba.reshape(1, 1))
